# baseline (speedup 1.0000x reference)
import functools

import jax
import jax.numpy as jnp
from jax import lax
from jax.experimental import pallas as pl
from jax.experimental.pallas import tpu as pltpu
from jax.experimental.pallas import tpu_sc as plsc

_VOCAB = 100000
_D = 300
_DP = 304
_B = 4096
_S = 200
_N = _B * _S
_NC = 2
_NS = 16
_NW = _NC * _NS
_PER_W = _N // _NW
_CH = 128
_N_CH = _PER_W // _CH


def _gather_body(idx_hbm, table_hbm, out_hbm, idx_v, rows_v, sem):
    wid = lax.axis_index("s") * _NC + lax.axis_index("c")
    base = wid * _PER_W

    def chunk(i, carry):
        off = base + i * _CH
        pltpu.sync_copy(idx_hbm.at[pl.ds(off, _CH)], idx_v)
        pltpu.async_copy(table_hbm.at[idx_v], rows_v, sem).wait()
        pltpu.sync_copy(rows_v, out_hbm.at[pl.ds(off, _CH)])
        return carry

    lax.fori_loop(0, _N_CH, chunk, 0)


@functools.partial(jax.jit, static_argnums=())
def kernel(data, table):
    idx = data.reshape(_N).astype(jnp.int32)
    table_p = jnp.pad(table, ((0, 0), (0, _DP - _D)))
    mesh = plsc.VectorSubcoreMesh(
        core_axis_name="c", subcore_axis_name="s",
        num_cores=_NC, num_subcores=_NS)
    k = pl.kernel(
        _gather_body,
        out_type=jax.ShapeDtypeStruct((_N, _DP), jnp.float32),
        mesh=mesh,
        compiler_params=pltpu.CompilerParams(use_tc_tiling_on_sc=False),
        scratch_types=[
            pltpu.VMEM((_CH,), jnp.int32),
            pltpu.VMEM((_CH, _DP), jnp.float32),
            pltpu.SemaphoreType.DMA,
        ],
    )
    out = k(idx, table_p)
    return out[:, :_D].reshape(_B, _S, _D)


# speedup vs baseline: 1.7599x; 1.7599x over previous
import functools

import jax
import jax.numpy as jnp
from jax import lax
from jax.experimental import pallas as pl
from jax.experimental.pallas import tpu as pltpu
from jax.experimental.pallas import tpu_sc as plsc

_VOCAB = 100000
_D = 300
_DA = 256
_DB = _D - _DA
_B = 4096
_S = 200
_N = _B * _S
_NC = 2
_NS = 16
_NW = _NC * _NS
_PER_W = _N // _NW
_CH = 128
_N_CH = _PER_W // _CH


def _gather_body(idx_hbm, ta_hbm, tb_hbm, out_hbm,
                 idx_v, rows_a, rows_b, tail_v, sem):
    wid = lax.axis_index("s") * _NC + lax.axis_index("c")
    base = wid * _PER_W

    def chunk(i, carry):
        off = base + i * _CH
        pltpu.sync_copy(idx_hbm.at[pl.ds(off, _CH)], idx_v)
        ca = pltpu.async_copy(ta_hbm.at[idx_v], rows_a, sem)
        cb = pltpu.async_copy(tb_hbm.at[idx_v], rows_b, sem)
        ca.wait()
        cb.wait()

        def row(r, c):
            tail_v[r, pl.ds(0, 16)] = rows_b[r, pl.ds(0, 16)]
            tail_v[r, pl.ds(16, 16)] = rows_b[r, pl.ds(16, 16)]
            tail_v[r, pl.ds(28, 16)] = rows_b[r, pl.ds(28, 16)]
            return c

        lax.fori_loop(0, _CH, row, 0)
        pltpu.sync_copy(rows_a, out_hbm.at[pl.ds(off, _CH), pl.ds(0, _DA)])
        pltpu.sync_copy(tail_v, out_hbm.at[pl.ds(off, _CH), pl.ds(_DA, _DB)])
        return carry

    lax.fori_loop(0, _N_CH, chunk, 0)


@functools.partial(jax.jit, static_argnums=())
def kernel(data, table):
    idx = data.reshape(_N).astype(jnp.int32)
    table_a = table[:, :_DA]
    table_b = jnp.pad(table[:, _DA:], ((0, 0), (0, 128 - _DB)))
    mesh = plsc.VectorSubcoreMesh(
        core_axis_name="c", subcore_axis_name="s",
        num_cores=_NC, num_subcores=_NS)
    k = pl.kernel(
        _gather_body,
        out_type=jax.ShapeDtypeStruct((_N, _D), jnp.float32),
        mesh=mesh,
        scratch_types=[
            pltpu.VMEM((_CH,), jnp.int32),
            pltpu.VMEM((_CH, _DA), jnp.float32),
            pltpu.VMEM((_CH, 128), jnp.float32),
            pltpu.VMEM((_CH, _DB), jnp.float32),
            pltpu.SemaphoreType.DMA,
        ],
    )
    out = k(idx, table_a, table_b)
    return out.reshape(_B, _S, _D)


# speedup vs baseline: 1.7786x; 1.0107x over previous
import functools

import jax
import jax.numpy as jnp
from jax import lax
from jax.experimental import pallas as pl
from jax.experimental.pallas import tpu as pltpu
from jax.experimental.pallas import tpu_sc as plsc

_VOCAB = 100000
_D = 300
_DA = 256
_DB = _D - _DA
_B = 4096
_S = 200
_N = _B * _S
_NC = 2
_NS = 16
_NW = _NC * _NS
_PER_W = _N // _NW
_CH = 128
_N_CH = _PER_W // _CH


def _gather_body(idx_hbm, tab_hbm, tb_hbm, out_hbm,
                 idx_v, rows_a, rows_b, tail_v, sem):
    wid = lax.axis_index("s") * _NC + lax.axis_index("c")
    base = wid * _PER_W

    def chunk(i, carry):
        off = base + i * _CH
        pltpu.sync_copy(idx_hbm.at[pl.ds(off, _CH)], idx_v)
        ca = pltpu.async_copy(tab_hbm.at[idx_v, pl.ds(0, _DA)], rows_a, sem)
        cb = pltpu.async_copy(tb_hbm.at[idx_v], rows_b, sem)
        ca.wait()
        cb.wait()

        def row(r, c):
            tail_v[r, pl.ds(0, 16)] = rows_b[r, pl.ds(0, 16)]
            tail_v[r, pl.ds(16, 16)] = rows_b[r, pl.ds(16, 16)]
            tail_v[r, pl.ds(28, 16)] = rows_b[r, pl.ds(28, 16)]
            return c

        lax.fori_loop(0, _CH, row, 0)
        pltpu.sync_copy(rows_a, out_hbm.at[pl.ds(off, _CH), pl.ds(0, _DA)])
        pltpu.sync_copy(tail_v, out_hbm.at[pl.ds(off, _CH), pl.ds(_DA, _DB)])
        return carry

    lax.fori_loop(0, _N_CH, chunk, 0)


@functools.partial(jax.jit, static_argnums=())
def kernel(data, table):
    idx = data.reshape(_N).astype(jnp.int32)
    table_b = jnp.pad(table[:, _DA:], ((0, 0), (0, 128 - _DB)))
    mesh = plsc.VectorSubcoreMesh(
        core_axis_name="c", subcore_axis_name="s",
        num_cores=_NC, num_subcores=_NS)
    k = pl.kernel(
        _gather_body,
        out_type=jax.ShapeDtypeStruct((_N, _D), jnp.float32),
        mesh=mesh,
        scratch_types=[
            pltpu.VMEM((_CH,), jnp.int32),
            pltpu.VMEM((_CH, _DA), jnp.float32),
            pltpu.VMEM((_CH, 128), jnp.float32),
            pltpu.VMEM((_CH, _DB), jnp.float32),
            pltpu.SemaphoreType.DMA,
        ],
    )
    out = k(idx, table, table_b)
    return out.reshape(_B, _S, _D)
